# baseline (speedup 1.0000x reference)
import functools

import jax
import jax.numpy as jnp
from jax import lax
from jax.experimental import pallas as pl
from jax.experimental.pallas import tpu as pltpu
from jax.experimental.pallas import tpu_sc as plsc

VOCAB = 100000
EMBED_DIM = 64
RANK = 128
SCALE = 2.0


_MERGE_BLK = 4000


def _merge_body(scale_ref, w_ref, a_ref, b_ref, out_ref):
    delta = jnp.dot(a_ref[...], b_ref[...], preferred_element_type=jnp.float32)
    merged = w_ref[...] + scale_ref[0] * delta
    m3 = merged.reshape(_MERGE_BLK // 2, 2, EMBED_DIM)
    out_ref[:, 0:EMBED_DIM] = m3[:, 0, :]
    out_ref[:, EMBED_DIM:2 * EMBED_DIM] = m3[:, 1, :]


def _merged_table(weight, lora_A, lora_B, scale):
    grid = VOCAB // _MERGE_BLK
    return pl.pallas_call(
        _merge_body,
        grid=(grid,),
        in_specs=[
            pl.BlockSpec(memory_space=pltpu.SMEM),
            pl.BlockSpec((_MERGE_BLK, EMBED_DIM), lambda i: (i, 0)),
            pl.BlockSpec((_MERGE_BLK, RANK), lambda i: (i, 0)),
            pl.BlockSpec((RANK, EMBED_DIM), lambda i: (0, 0)),
        ],
        out_specs=pl.BlockSpec((_MERGE_BLK // 2, 2 * EMBED_DIM), lambda i: (i, 0)),
        out_shape=jax.ShapeDtypeStruct((VOCAB // 2, 2 * EMBED_DIM), jnp.float32),
    )(scale, weight, lora_A, lora_B)


_NC, _NS = 2, 16
_NW = _NC * _NS
_B = 4096 * 200
_B_PER_W = _B // _NW
_CHUNK = 512
_NCHUNK = _B_PER_W // _CHUNK
_NPAIR = _NCHUNK // 2


def _gather_body(table_hbm, idx_hbm, out_hbm,
                 idx_v0, idx_v1, rows_v0, rows_v1, sem0, sem1):
    wid = lax.axis_index("s") * _NC + lax.axis_index("c")
    base = wid * _B_PER_W

    def load_idx(i, iv):
        pltpu.sync_copy(idx_hbm.at[pl.ds(base + i * _CHUNK, _CHUNK)], iv)

    def start_gather(iv, rv, sem):
        return pltpu.async_copy(table_hbm.at[iv], rv, sem)

    def write_out(i, rv):
        pltpu.sync_copy(
            rv, out_hbm.at[pl.ds(base + i * _CHUNK, _CHUNK), pl.ds(0, EMBED_DIM)])

    load_idx(0, idx_v0)
    start_gather(idx_v0, rows_v0, sem0)

    def pair(j, carry):
        i0 = 2 * j

        load_idx(i0 + 1, idx_v1)
        pltpu.make_async_copy(table_hbm.at[idx_v0], rows_v0, sem0).wait()
        start_gather(idx_v1, rows_v1, sem1)
        write_out(i0, rows_v0)

        @pl.when(j < _NPAIR - 1)
        def _():
            load_idx(i0 + 2, idx_v0)
        pltpu.make_async_copy(table_hbm.at[idx_v1], rows_v1, sem1).wait()

        @pl.when(j < _NPAIR - 1)
        def _():
            start_gather(idx_v0, rows_v0, sem0)
        write_out(i0 + 1, rows_v1)
        return carry

    lax.fori_loop(0, _NPAIR, pair, 0)


@functools.lru_cache(maxsize=1)
def _make_gather():
    return pl.kernel(
        _gather_body,
        out_type=jax.ShapeDtypeStruct((_B, 2 * EMBED_DIM), jnp.float32),
        mesh=plsc.VectorSubcoreMesh(core_axis_name="c", subcore_axis_name="s"),
        scratch_types=[
            pltpu.VMEM((_CHUNK,), jnp.int32),
            pltpu.VMEM((_CHUNK,), jnp.int32),
            pltpu.VMEM((_CHUNK, EMBED_DIM), jnp.float32),
            pltpu.VMEM((_CHUNK, EMBED_DIM), jnp.float32),
            pltpu.SemaphoreType.DMA,
            pltpu.SemaphoreType.DMA,
        ],
        compiler_params=pltpu.CompilerParams(use_tc_tiling_on_sc=False),
    )


def kernel(x, enabled_lora, weight, lora_A, lora_B):
    scale = jnp.where(enabled_lora, jnp.float32(SCALE), jnp.float32(0.0))
    table = _merged_table(weight, lora_A, lora_B, scale.reshape(1))
    out = _make_gather()(table.reshape(VOCAB, EMBED_DIM), x.reshape(-1))
    return out[:, :EMBED_DIM].reshape(x.shape + (EMBED_DIM,))


# speedup vs baseline: 1.0201x; 1.0201x over previous
import functools

import jax
import jax.numpy as jnp
from jax import lax
from jax.experimental import pallas as pl
from jax.experimental.pallas import tpu as pltpu
from jax.experimental.pallas import tpu_sc as plsc

VOCAB = 100000
EMBED_DIM = 64
RANK = 128
SCALE = 2.0


_MERGE_BLK = 10000


def _merge_body(scale_ref, w_ref, a_ref, b_ref, out_ref):
    delta = jnp.dot(a_ref[...], b_ref[...], preferred_element_type=jnp.float32)
    merged = w_ref[...] + scale_ref[0] * delta
    m3 = merged.reshape(_MERGE_BLK // 2, 2, EMBED_DIM)
    out_ref[:, 0:EMBED_DIM] = m3[:, 0, :]
    out_ref[:, EMBED_DIM:2 * EMBED_DIM] = m3[:, 1, :]


def _merged_table(weight, lora_A, lora_B, scale):
    grid = VOCAB // _MERGE_BLK
    return pl.pallas_call(
        _merge_body,
        grid=(grid,),
        in_specs=[
            pl.BlockSpec(memory_space=pltpu.SMEM),
            pl.BlockSpec((_MERGE_BLK, EMBED_DIM), lambda i: (i, 0)),
            pl.BlockSpec((_MERGE_BLK, RANK), lambda i: (i, 0)),
            pl.BlockSpec((RANK, EMBED_DIM), lambda i: (0, 0)),
        ],
        out_specs=pl.BlockSpec((_MERGE_BLK // 2, 2 * EMBED_DIM), lambda i: (i, 0)),
        out_shape=jax.ShapeDtypeStruct((VOCAB // 2, 2 * EMBED_DIM), jnp.float32),
    )(scale, weight, lora_A, lora_B)


_NC, _NS = 2, 16
_NW = _NC * _NS
_B = 4096 * 200
_B_PER_W = _B // _NW
_CHUNK = 512
_NCHUNK = _B_PER_W // _CHUNK
_NPAIR = _NCHUNK // 2


def _gather_body(table_hbm, idx_hbm, out_hbm,
                 idx_v0, idx_v1, rows_v0, rows_v1, sem0, sem1):
    wid = lax.axis_index("s") * _NC + lax.axis_index("c")
    base = wid * _B_PER_W

    def load_idx(i, iv):
        pltpu.sync_copy(idx_hbm.at[pl.ds(base + i * _CHUNK, _CHUNK)], iv)

    def start_gather(iv, rv, sem):
        return pltpu.async_copy(table_hbm.at[iv], rv, sem)

    def write_out(i, rv):
        pltpu.sync_copy(
            rv, out_hbm.at[pl.ds(base + i * _CHUNK, _CHUNK), pl.ds(0, EMBED_DIM)])

    load_idx(0, idx_v0)
    start_gather(idx_v0, rows_v0, sem0)

    def pair(j, carry):
        i0 = 2 * j

        load_idx(i0 + 1, idx_v1)
        pltpu.make_async_copy(table_hbm.at[idx_v0], rows_v0, sem0).wait()
        start_gather(idx_v1, rows_v1, sem1)
        write_out(i0, rows_v0)

        @pl.when(j < _NPAIR - 1)
        def _():
            load_idx(i0 + 2, idx_v0)
        pltpu.make_async_copy(table_hbm.at[idx_v1], rows_v1, sem1).wait()

        @pl.when(j < _NPAIR - 1)
        def _():
            start_gather(idx_v0, rows_v0, sem0)
        write_out(i0 + 1, rows_v1)
        return carry

    lax.fori_loop(0, _NPAIR, pair, 0)


@functools.lru_cache(maxsize=1)
def _make_gather():
    return pl.kernel(
        _gather_body,
        out_type=jax.ShapeDtypeStruct((_B, 2 * EMBED_DIM), jnp.float32),
        mesh=plsc.VectorSubcoreMesh(core_axis_name="c", subcore_axis_name="s"),
        scratch_types=[
            pltpu.VMEM((_CHUNK,), jnp.int32),
            pltpu.VMEM((_CHUNK,), jnp.int32),
            pltpu.VMEM((_CHUNK, EMBED_DIM), jnp.float32),
            pltpu.VMEM((_CHUNK, EMBED_DIM), jnp.float32),
            pltpu.SemaphoreType.DMA,
            pltpu.SemaphoreType.DMA,
        ],
        compiler_params=pltpu.CompilerParams(use_tc_tiling_on_sc=False),
    )


def kernel(x, enabled_lora, weight, lora_A, lora_B):
    scale = jnp.where(enabled_lora, jnp.float32(SCALE), jnp.float32(0.0))
    table = _merged_table(weight, lora_A, lora_B, scale.reshape(1))
    out = _make_gather()(table.reshape(VOCAB, EMBED_DIM), x.reshape(-1))
    return out[:, :EMBED_DIM].reshape(x.shape + (EMBED_DIM,))
